# Initial kernel scaffold; baseline (speedup 1.0000x reference)
import jax
import jax.numpy as jnp
from jax.experimental import pallas as pl


def kernel(indices, lin_tables, femb, bias):
    raise NotImplementedError("write your pallas kernel here")


# speedup vs baseline: 20.7096x; 20.7096x over previous
import functools

import jax
import jax.numpy as jnp
import numpy as np
from jax import lax
from jax.experimental import pallas as pl
from jax.experimental.pallas import tpu as pltpu
from jax.experimental.pallas import tpu_sc as plsc

F = 26
B = 4096
V = 1000
D = 16

NC = 2
NS = 16
NW = NC * NS
BW = B // NW
P = 13
NPAIR = (F * (F - 1)) // 2
NBLK = NPAIR // P
META_COLS = 4 * NPAIR + 16


def _build_meta() -> np.ndarray:
    meta = np.zeros((META_COLS,), dtype=np.int32)
    p = 0
    for i in range(F - 1):
        for j in range(i + 1, F):
            meta[4 * p + 0] = i
            meta[4 * p + 1] = (i * F + j) * V
            meta[4 * p + 2] = j
            meta[4 * p + 3] = (j * F + i) * V
            p += 1
    return meta


_META = _build_meta()


def _ffm_body(idx_hbm, femb_hbm, lin_hbm, meta_hbm, bias_hbm, out_hbm,
              idxbuf, metabuf, biasbuf, offsA, offsB, bufA, bufB,
              acc, res, sem):
    wid = lax.axis_index("s") * NC + lax.axis_index("c")
    base = wid * BW

    for f in range(F):
        pltpu.sync_copy(idx_hbm.at[f, pl.ds(base, BW)], idxbuf.at[f])
    pltpu.sync_copy(meta_hbm, metabuf)
    pltpu.sync_copy(bias_hbm, biasbuf)

    zero16 = jnp.zeros((16,), jnp.float32)

    @pl.loop(0, BW)
    def _zero(b):
        acc[pl.ds(b * D, D)] = zero16

    @pl.loop(0, NBLK)
    def _block(bb):
        copies = []
        for q in range(P):
            p = bb * P + q
            m = metabuf[pl.ds(4 * p, 16)]
            iA = m[0]
            bA = m[1]
            iB = m[2]
            bB = m[3]
            for c in range(BW // 16):
                sl = pl.ds(c * 16, 16)
                offsA[q, sl] = idxbuf[iA, sl] + bA
                offsB[q, sl] = idxbuf[iB, sl] + bB
            copies.append(pltpu.async_copy(
                femb_hbm.at[offsA.at[q]], bufA.at[q], sem))
            copies.append(pltpu.async_copy(
                femb_hbm.at[offsB.at[q]], bufB.at[q], sem))
        for cp in copies:
            cp.wait()

        @pl.loop(0, BW)
        def _mac(b):
            a = acc[pl.ds(b * D, D)]
            for q in range(P):
                a = a + bufA[q, b, :] * bufB[q, b, :]
            acc[pl.ds(b * D, D)] = a

    @pl.loop(0, F)
    def _lin(f):
        for c in range(BW // 16):
            sl = pl.ds(c * 16, 16)
            offsA[0, sl] = idxbuf[f, sl] + f * V
        pltpu.async_copy(lin_hbm.at[offsA.at[0]], bufA.at[0], sem).wait()

        @pl.loop(0, BW)
        def _acc_lin(b):
            acc[pl.ds(b * D, D)] = acc[pl.ds(b * D, D)] + bufA[0, b, :]

    lane = lax.iota(jnp.int32, 16)

    @pl.loop(0, BW // 16)
    def _final(c):
        x = biasbuf[:]
        for k in range(16):
            a = acc[pl.ds((c * 16 + k) * D, D)]
            s = jnp.zeros((16,), jnp.float32)
            for d in range(D):
                s = s + jnp.full((16,), a[d], jnp.float32)
            x = x + jnp.where(lane == k, s, 0.0)
        res[pl.ds(c * 16, 16)] = 1.0 / (1.0 + jnp.exp(-x))

    pltpu.sync_copy(res, out_hbm.at[pl.ds(base, BW)])


@jax.jit
def _ffm_call(indices, femb_flat, lin_pad, meta, bias16):
    mesh = plsc.VectorSubcoreMesh(core_axis_name="c", subcore_axis_name="s")
    return pl.kernel(
        _ffm_body,
        out_type=jax.ShapeDtypeStruct((B,), jnp.float32),
        mesh=mesh,
        compiler_params=pltpu.CompilerParams(use_tc_tiling_on_sc=False),
        scratch_types=[
            pltpu.VMEM((F, BW), jnp.int32),
            pltpu.VMEM((META_COLS,), jnp.int32),
            pltpu.VMEM((16,), jnp.float32),
            pltpu.VMEM((P, BW), jnp.int32),
            pltpu.VMEM((P, BW), jnp.int32),
            pltpu.VMEM((P, BW, D), jnp.float32),
            pltpu.VMEM((P, BW, D), jnp.float32),
            pltpu.VMEM((BW * D,), jnp.float32),
            pltpu.VMEM((BW,), jnp.float32),
            pltpu.SemaphoreType.DMA,
        ],
    )(indices, femb_flat, lin_pad, meta, bias16)


def kernel(indices, lin_tables, femb, bias):
    femb_flat = femb.reshape(F * F * V, D)
    lin_pad = (jnp.zeros((F, V, D), jnp.float32)
               .at[:, :, 0].set(lin_tables).reshape(F * V, D))
    bias16 = jnp.broadcast_to(bias.astype(jnp.float32), (16,))
    return _ffm_call(indices, femb_flat, lin_pad, jnp.asarray(_META), bias16)


# speedup vs baseline: 20.8090x; 1.0048x over previous
import functools

import jax
import jax.numpy as jnp
import numpy as np
from jax import lax
from jax.experimental import pallas as pl
from jax.experimental.pallas import tpu as pltpu
from jax.experimental.pallas import tpu_sc as plsc

F = 26
B = 4096
V = 1000
D = 16

NC = 2
NS = 16
NW = NC * NS
BW = B // NW
P = 13
NPAIR = (F * (F - 1)) // 2
NBLK = NPAIR // P
META_COLS = 4 * NPAIR + 16


def _build_meta() -> np.ndarray:
    meta = np.zeros((META_COLS,), dtype=np.int32)
    p = 0
    for i in range(F - 1):
        for j in range(i + 1, F):
            meta[4 * p + 0] = i
            meta[4 * p + 1] = j
            p += 1
    return meta


_META = _build_meta()


def _ffm_body(idx_hbm, femb_hbm, lin_hbm, meta_hbm, bias_hbm, out_hbm,
              idxbuf, metabuf, biasbuf, bufA, bufB, acc, res, sem):
    wid = lax.axis_index("s") * NC + lax.axis_index("c")
    base = wid * BW

    for f in range(F):
        pltpu.sync_copy(idx_hbm.at[f, pl.ds(base, BW)], idxbuf.at[f])
    pltpu.sync_copy(meta_hbm, metabuf)
    pltpu.sync_copy(bias_hbm, biasbuf)

    zero16 = jnp.zeros((16,), jnp.float32)

    @pl.loop(0, BW)
    def _zero(b):
        acc[pl.ds(b * D, D)] = zero16

    @pl.loop(0, NBLK)
    def _block(bb):
        copies = []
        for q in range(P):
            p = bb * P + q
            m = metabuf[pl.ds(4 * p, 16)]
            iA = m[0]
            jA = m[1]
            copies.append(pltpu.async_copy(
                femb_hbm.at[iA, jA].at[idxbuf.at[iA]], bufA.at[q], sem))
            copies.append(pltpu.async_copy(
                femb_hbm.at[jA, iA].at[idxbuf.at[jA]], bufB.at[q], sem))
        for cp in copies:
            cp.wait()

        @pl.loop(0, BW)
        def _mac(b):
            a = acc[pl.ds(b * D, D)]
            for q in range(P):
                a = a + bufA[q, b, :] * bufB[q, b, :]
            acc[pl.ds(b * D, D)] = a

    @pl.loop(0, F)
    def _lin(f):
        pltpu.async_copy(lin_hbm.at[f].at[idxbuf.at[f]], bufA.at[0],
                         sem).wait()

        @pl.loop(0, BW)
        def _acc_lin(b):
            acc[pl.ds(b * D, D)] = acc[pl.ds(b * D, D)] + bufA[0, b, :]

    lane = lax.iota(jnp.int32, 16)

    @pl.loop(0, BW // 16)
    def _final(c):
        x = biasbuf[:]
        for k in range(16):
            a = acc[pl.ds((c * 16 + k) * D, D)]
            s = jnp.zeros((16,), jnp.float32)
            for d in range(D):
                s = s + jnp.full((16,), a[d], jnp.float32)
            x = x + jnp.where(lane == k, s, 0.0)
        res[pl.ds(c * 16, 16)] = 1.0 / (1.0 + jnp.exp(-x))

    pltpu.sync_copy(res, out_hbm.at[pl.ds(base, BW)])


@jax.jit
def _ffm_call(indices, femb_flat, lin_pad, meta, bias16):
    mesh = plsc.VectorSubcoreMesh(core_axis_name="c", subcore_axis_name="s")
    return pl.kernel(
        _ffm_body,
        out_type=jax.ShapeDtypeStruct((B,), jnp.float32),
        mesh=mesh,
        compiler_params=pltpu.CompilerParams(use_tc_tiling_on_sc=False),
        scratch_types=[
            pltpu.VMEM((F, BW), jnp.int32),
            pltpu.VMEM((META_COLS,), jnp.int32),
            pltpu.VMEM((16,), jnp.float32),
            pltpu.VMEM((P, BW, D), jnp.float32),
            pltpu.VMEM((P, BW, D), jnp.float32),
            pltpu.VMEM((BW * D,), jnp.float32),
            pltpu.VMEM((BW,), jnp.float32),
            pltpu.SemaphoreType.DMA,
        ],
    )(indices, femb_flat, lin_pad, meta, bias16)


def kernel(indices, lin_tables, femb, bias):
    lin_pad = (jnp.zeros((F, V, D), jnp.float32)
               .at[:, :, 0].set(lin_tables))
    bias16 = jnp.broadcast_to(bias.astype(jnp.float32), (16,))
    return _ffm_call(indices, femb, lin_pad, jnp.asarray(_META), bias16)


# speedup vs baseline: 24.6648x; 1.1853x over previous
import functools

import jax
import jax.numpy as jnp
import numpy as np
from jax import lax
from jax.experimental import pallas as pl
from jax.experimental.pallas import tpu as pltpu
from jax.experimental.pallas import tpu_sc as plsc

F = 26
B = 4096
V = 1000
D = 16

NC = 2
NS = 16
NW = NC * NS
BW = B // NW
P = 13
NPAIR = (F * (F - 1)) // 2
NBLK = NPAIR // P
META_COLS = 4 * NPAIR + 16


def _build_meta() -> np.ndarray:
    meta = np.zeros((META_COLS,), dtype=np.int32)
    p = 0
    for i in range(F - 1):
        for j in range(i + 1, F):
            meta[4 * p + 0] = i
            meta[4 * p + 1] = j
            p += 1
    return meta


_META = _build_meta()


def _ffm_body(idx_hbm, femb_hbm, lin_hbm, meta_hbm, bias_hbm, out_hbm,
              idxbuf, metabuf, biasbuf, bufA0, bufB0, bufA1, bufB1,
              acc, res, sem0, sem1):
    wid = lax.axis_index("s") * NC + lax.axis_index("c")
    base = wid * BW

    for f in range(F):
        pltpu.sync_copy(idx_hbm.at[f, pl.ds(base, BW)], idxbuf.at[f])
    pltpu.sync_copy(meta_hbm, metabuf)
    pltpu.sync_copy(bias_hbm, biasbuf)

    zero16 = jnp.zeros((16,), jnp.float32)

    @pl.loop(0, BW)
    def _zero(b):
        acc[pl.ds(b * D, D)] = zero16

    def _fire(bb, bA, bB, sem):
        for q in range(P):
            p = bb * P + q
            m = metabuf[pl.ds(4 * p, 16)]
            iA = m[0]
            jA = m[1]
            pltpu.async_copy(
                femb_hbm.at[iA, jA].at[idxbuf.at[iA]], bA.at[q], sem)
            pltpu.async_copy(
                femb_hbm.at[jA, iA].at[idxbuf.at[jA]], bB.at[q], sem)

    def _drain(bA, bB, sem):
        for q in range(P):
            pltpu.make_async_copy(
                femb_hbm.at[0, 1].at[idxbuf.at[0]], bA.at[q], sem).wait()
            pltpu.make_async_copy(
                femb_hbm.at[0, 1].at[idxbuf.at[0]], bB.at[q], sem).wait()

    def _mac(bA, bB):
        @pl.loop(0, BW)
        def _m(b):
            a = acc[pl.ds(b * D, D)]
            for q in range(P):
                a = a + bA[q, b, :] * bB[q, b, :]
            acc[pl.ds(b * D, D)] = a

    _fire(0, bufA0, bufB0, sem0)

    @pl.loop(0, (NBLK - 1) // 2)
    def _piped(k):
        bb = k * 2
        _fire(bb + 1, bufA1, bufB1, sem1)
        _drain(bufA0, bufB0, sem0)
        _mac(bufA0, bufB0)
        _fire(bb + 2, bufA0, bufB0, sem0)
        _drain(bufA1, bufB1, sem1)
        _mac(bufA1, bufB1)

    _drain(bufA0, bufB0, sem0)
    _mac(bufA0, bufB0)

    def _lin_add(b0):
        @pl.loop(0, BW)
        def _acc_lin(b):
            acc[pl.ds(b * D, D)] = acc[pl.ds(b * D, D)] + b0[0, b, :]

    @pl.loop(0, F // 2)
    def _lin(k):
        f = k * 2
        pltpu.async_copy(lin_hbm.at[f].at[idxbuf.at[f]], bufA0.at[0], sem0)
        pltpu.async_copy(lin_hbm.at[f + 1].at[idxbuf.at[f + 1]],
                         bufA1.at[0], sem1)
        pltpu.make_async_copy(
            lin_hbm.at[0].at[idxbuf.at[0]], bufA0.at[0], sem0).wait()
        _lin_add(bufA0)
        pltpu.make_async_copy(
            lin_hbm.at[0].at[idxbuf.at[0]], bufA1.at[0], sem1).wait()
        _lin_add(bufA1)

    lane = lax.iota(jnp.int32, 16)

    @pl.loop(0, BW // 16)
    def _final(c):
        x = biasbuf[:]
        for k in range(16):
            a = acc[pl.ds((c * 16 + k) * D, D)]
            s = jnp.zeros((16,), jnp.float32)
            for d in range(D):
                s = s + jnp.full((16,), a[d], jnp.float32)
            x = x + jnp.where(lane == k, s, 0.0)
        res[pl.ds(c * 16, 16)] = 1.0 / (1.0 + jnp.exp(-x))

    pltpu.sync_copy(res, out_hbm.at[pl.ds(base, BW)])


@jax.jit
def _ffm_call(indices, femb_flat, lin_pad, meta, bias16):
    mesh = plsc.VectorSubcoreMesh(core_axis_name="c", subcore_axis_name="s")
    return pl.kernel(
        _ffm_body,
        out_type=jax.ShapeDtypeStruct((B,), jnp.float32),
        mesh=mesh,
        compiler_params=pltpu.CompilerParams(use_tc_tiling_on_sc=False),
        scratch_types=[
            pltpu.VMEM((F, BW), jnp.int32),
            pltpu.VMEM((META_COLS,), jnp.int32),
            pltpu.VMEM((16,), jnp.float32),
            pltpu.VMEM((P, BW, D), jnp.float32),
            pltpu.VMEM((P, BW, D), jnp.float32),
            pltpu.VMEM((P, BW, D), jnp.float32),
            pltpu.VMEM((P, BW, D), jnp.float32),
            pltpu.VMEM((BW * D,), jnp.float32),
            pltpu.VMEM((BW,), jnp.float32),
            pltpu.SemaphoreType.DMA,
            pltpu.SemaphoreType.DMA,
        ],
    )(indices, femb_flat, lin_pad, meta, bias16)


def kernel(indices, lin_tables, femb, bias):
    lin_pad = (jnp.zeros((F, V, D), jnp.float32)
               .at[:, :, 0].set(lin_tables))
    bias16 = jnp.broadcast_to(bias.astype(jnp.float32), (16,))
    return _ffm_call(indices, femb, lin_pad, jnp.asarray(_META), bias16)


# speedup vs baseline: 26.4804x; 1.0736x over previous
import functools

import jax
import jax.numpy as jnp
import numpy as np
from jax import lax
from jax.experimental import pallas as pl
from jax.experimental.pallas import tpu as pltpu
from jax.experimental.pallas import tpu_sc as plsc

F = 26
B = 4096
V = 1000
D = 16

NC = 2
NS = 16
NW = NC * NS
BW = B // NW
P = 13
NPAIR = (F * (F - 1)) // 2
NBLK = NPAIR // P
META_COLS = 4 * NPAIR + 16


def _build_meta() -> np.ndarray:
    meta = np.zeros((META_COLS,), dtype=np.int32)
    p = 0
    for i in range(F - 1):
        for j in range(i + 1, F):
            meta[4 * p + 0] = i
            meta[4 * p + 1] = j
            p += 1
    return meta


_META = _build_meta()


def _ffm_body(idx_hbm, femb_hbm, lin_hbm, meta_hbm, bias_hbm, out_hbm,
              idxbuf, metabuf, biasbuf, bufA0, bufB0, bufA1, bufB1,
              lbuf0, lbuf1, lacc, acc, res, sem0, sem1):
    wid = lax.axis_index("s") * NC + lax.axis_index("c")
    base = wid * BW
    rot = jnp.where(wid >= NBLK, wid - NBLK, wid)
    rot13 = jnp.where(wid >= 26, wid - 26, jnp.where(wid >= 13, wid - 13, wid))

    for f in range(F):
        pltpu.sync_copy(idx_hbm.at[f, pl.ds(base, BW)], idxbuf.at[f])
    pltpu.sync_copy(meta_hbm, metabuf)
    pltpu.sync_copy(bias_hbm, biasbuf)

    zero16 = jnp.zeros((16,), jnp.float32)

    @pl.loop(0, BW)
    def _zero(b):
        acc[pl.ds(b * D, D)] = zero16

    def _fire(bb, bA, bB, sem):
        eb = bb + rot
        eb = jnp.where(eb >= NBLK, eb - NBLK, eb)
        for q in range(P):
            p = eb * P + q
            m = metabuf[pl.ds(4 * p, 16)]
            iA = m[0]
            jA = m[1]
            pltpu.async_copy(
                femb_hbm.at[iA, jA].at[idxbuf.at[iA]], bA.at[q], sem)
            pltpu.async_copy(
                femb_hbm.at[jA, iA].at[idxbuf.at[jA]], bB.at[q], sem)

    def _drain(bA, bB, sem):
        for q in range(P):
            pltpu.make_async_copy(
                femb_hbm.at[0, 1].at[idxbuf.at[0]], bA.at[q], sem).wait()
            pltpu.make_async_copy(
                femb_hbm.at[0, 1].at[idxbuf.at[0]], bB.at[q], sem).wait()

    def _mac(bA, bB):
        @pl.loop(0, BW)
        def _m(b):
            a = acc[pl.ds(b * D, D)]
            for q in range(P):
                a = a + bA[q, b, :] * bB[q, b, :]
            acc[pl.ds(b * D, D)] = a

    _fire(0, bufA0, bufB0, sem0)

    @pl.loop(0, (NBLK - 1) // 2)
    def _piped(k):
        bb = k * 2
        _fire(bb + 1, bufA1, bufB1, sem1)
        _drain(bufA0, bufB0, sem0)
        _mac(bufA0, bufB0)
        _fire(bb + 2, bufA0, bufB0, sem0)
        _drain(bufA1, bufB1, sem1)
        _mac(bufA1, bufB1)

    _drain(bufA0, bufB0, sem0)
    _mac(bufA0, bufB0)

    zero16f = jnp.zeros((16,), jnp.float32)
    for c in range(BW // 16):
        lacc[pl.ds(c * 16, 16)] = zero16f

    def _lin_add(lb):
        for c in range(BW // 16):
            sl = pl.ds(c * 16, 16)
            lacc[sl] = lacc[sl] + lb[sl]

    @pl.loop(0, F // 2)
    def _lin(k):
        kr = k + rot13
        kr = jnp.where(kr >= F // 2, kr - F // 2, kr)
        f = kr * 2
        pltpu.async_copy(lin_hbm.at[f].at[idxbuf.at[f]], lbuf0, sem0)
        pltpu.async_copy(lin_hbm.at[f + 1].at[idxbuf.at[f + 1]], lbuf1, sem1)
        pltpu.make_async_copy(
            lin_hbm.at[0].at[idxbuf.at[0]], lbuf0, sem0).wait()
        _lin_add(lbuf0)
        pltpu.make_async_copy(
            lin_hbm.at[0].at[idxbuf.at[0]], lbuf1, sem1).wait()
        _lin_add(lbuf1)

    lane = lax.iota(jnp.int32, 16)

    @pl.loop(0, BW // 16)
    def _final(c):
        x = biasbuf[:] + lacc[pl.ds(c * 16, 16)]
        for k in range(16):
            a = acc[pl.ds((c * 16 + k) * D, D)]
            s = jnp.zeros((16,), jnp.float32)
            for d in range(D):
                s = s + jnp.full((16,), a[d], jnp.float32)
            x = x + jnp.where(lane == k, s, 0.0)
        res[pl.ds(c * 16, 16)] = 1.0 / (1.0 + jnp.exp(-x))

    pltpu.sync_copy(res, out_hbm.at[pl.ds(base, BW)])


@jax.jit
def _ffm_call(indices, femb_flat, lin_pad, meta, bias16):
    mesh = plsc.VectorSubcoreMesh(core_axis_name="c", subcore_axis_name="s")
    return pl.kernel(
        _ffm_body,
        out_type=jax.ShapeDtypeStruct((B,), jnp.float32),
        mesh=mesh,
        compiler_params=pltpu.CompilerParams(use_tc_tiling_on_sc=False),
        scratch_types=[
            pltpu.VMEM((F, BW), jnp.int32),
            pltpu.VMEM((META_COLS,), jnp.int32),
            pltpu.VMEM((16,), jnp.float32),
            pltpu.VMEM((P, BW, D), jnp.float32),
            pltpu.VMEM((P, BW, D), jnp.float32),
            pltpu.VMEM((P, BW, D), jnp.float32),
            pltpu.VMEM((P, BW, D), jnp.float32),
            pltpu.VMEM((BW,), jnp.float32),
            pltpu.VMEM((BW,), jnp.float32),
            pltpu.VMEM((BW,), jnp.float32),
            pltpu.VMEM((BW * D,), jnp.float32),
            pltpu.VMEM((BW,), jnp.float32),
            pltpu.SemaphoreType.DMA,
            pltpu.SemaphoreType.DMA,
        ],
    )(indices, femb_flat, lin_pad, meta, bias16)


def kernel(indices, lin_tables, femb, bias):
    bias16 = jnp.broadcast_to(bias.astype(jnp.float32), (16,))
    return _ffm_call(indices, femb, lin_tables, jnp.asarray(_META), bias16)


# speedup vs baseline: 26.6845x; 1.0077x over previous
import functools

import jax
import jax.numpy as jnp
import numpy as np
from jax import lax
from jax.experimental import pallas as pl
from jax.experimental.pallas import tpu as pltpu
from jax.experimental.pallas import tpu_sc as plsc

F = 26
B = 4096
V = 1000
D = 16

NC = 2
NS = 16
NW = NC * NS
BW = B // NW
P = 13
NPAIR = (F * (F - 1)) // 2
NBLK = NPAIR // P
META_COLS = 4 * NPAIR + 16


def _build_meta() -> np.ndarray:
    meta = np.zeros((META_COLS,), dtype=np.int32)
    p = 0
    for i in range(F - 1):
        for j in range(i + 1, F):
            meta[4 * p + 0] = i
            meta[4 * p + 1] = j
            p += 1
    return meta


_META = _build_meta()


def _ffm_body(idx_hbm, femb_hbm, lin_hbm, meta_hbm, bias_hbm, out_hbm,
              idxbuf, metabuf, biasbuf, bufA0, bufB0, bufA1, bufB1,
              linbuf, lacc, acc, res, sem0, sem1, sem2):
    wid = lax.axis_index("s") * NC + lax.axis_index("c")
    base = wid * BW
    rot = jnp.where(wid >= NBLK, wid - NBLK, wid)

    for f in range(F):
        pltpu.sync_copy(idx_hbm.at[f, pl.ds(base, BW)], idxbuf.at[f])
    pltpu.sync_copy(meta_hbm, metabuf)
    pltpu.sync_copy(bias_hbm, biasbuf)

    zero16 = jnp.zeros((16,), jnp.float32)

    @pl.loop(0, BW)
    def _zero(b):
        acc[pl.ds(b * D, D)] = zero16

    def _fire(bb, bA, bB, sem):
        eb = bb + rot
        eb = jnp.where(eb >= NBLK, eb - NBLK, eb)
        for q in range(P):
            p = eb * P + q
            m = metabuf[pl.ds(4 * p, 16)]
            iA = m[0]
            jA = m[1]
            pltpu.async_copy(
                femb_hbm.at[iA, jA].at[idxbuf.at[iA]], bA.at[q], sem)
            pltpu.async_copy(
                femb_hbm.at[jA, iA].at[idxbuf.at[jA]], bB.at[q], sem)

    def _drain(bA, bB, sem):
        dummy = femb_hbm.at[0, pl.ds(0, P), pl.ds(0, BW), :]
        pltpu.make_async_copy(dummy, bA, sem).wait()
        pltpu.make_async_copy(dummy, bB, sem).wait()

    def _mac(bA, bB):
        @pl.loop(0, BW)
        def _m(b):
            a = acc[pl.ds(b * D, D)]
            for q in range(P):
                a = a + bA[q, b, :] * bB[q, b, :]
            acc[pl.ds(b * D, D)] = a

    for f in range(F):
        pltpu.async_copy(lin_hbm.at[f].at[idxbuf.at[f]], linbuf.at[f], sem2)

    _fire(0, bufA0, bufB0, sem0)

    @pl.loop(0, (NBLK - 1) // 2)
    def _piped(k):
        bb = k * 2
        _fire(bb + 1, bufA1, bufB1, sem1)
        _drain(bufA0, bufB0, sem0)
        _mac(bufA0, bufB0)
        _fire(bb + 2, bufA0, bufB0, sem0)
        _drain(bufA1, bufB1, sem1)
        _mac(bufA1, bufB1)

    _drain(bufA0, bufB0, sem0)
    _mac(bufA0, bufB0)

    pltpu.make_async_copy(lin_hbm.at[:, pl.ds(0, BW)], linbuf, sem2).wait()

    @pl.loop(0, BW // 16)
    def _linred(c):
        sl = pl.ds(c * 16, 16)
        v = linbuf[0, sl]
        for f in range(1, F):
            v = v + linbuf[f, sl]
        lacc[sl] = v

    lane = lax.iota(jnp.int32, 16)

    @pl.loop(0, BW // 16)
    def _final(c):
        x = biasbuf[:] + lacc[pl.ds(c * 16, 16)]
        for k in range(16):
            a = acc[pl.ds((c * 16 + k) * D, D)]
            s = jnp.zeros((16,), jnp.float32)
            for d in range(D):
                s = s + jnp.full((16,), a[d], jnp.float32)
            x = x + jnp.where(lane == k, s, 0.0)
        res[pl.ds(c * 16, 16)] = 1.0 / (1.0 + jnp.exp(-x))

    pltpu.sync_copy(res, out_hbm.at[pl.ds(base, BW)])


@jax.jit
def _ffm_call(indices, femb_flat, lin_pad, meta, bias16):
    mesh = plsc.VectorSubcoreMesh(core_axis_name="c", subcore_axis_name="s")
    return pl.kernel(
        _ffm_body,
        out_type=jax.ShapeDtypeStruct((B,), jnp.float32),
        mesh=mesh,
        compiler_params=pltpu.CompilerParams(use_tc_tiling_on_sc=False),
        scratch_types=[
            pltpu.VMEM((F, BW), jnp.int32),
            pltpu.VMEM((META_COLS,), jnp.int32),
            pltpu.VMEM((16,), jnp.float32),
            pltpu.VMEM((P, BW, D), jnp.float32),
            pltpu.VMEM((P, BW, D), jnp.float32),
            pltpu.VMEM((P, BW, D), jnp.float32),
            pltpu.VMEM((P, BW, D), jnp.float32),
            pltpu.VMEM((F, BW), jnp.float32),
            pltpu.VMEM((BW,), jnp.float32),
            pltpu.VMEM((BW * D,), jnp.float32),
            pltpu.VMEM((BW,), jnp.float32),
            pltpu.SemaphoreType.DMA,
            pltpu.SemaphoreType.DMA,
            pltpu.SemaphoreType.DMA,
        ],
    )(indices, femb_flat, lin_pad, meta, bias16)


def kernel(indices, lin_tables, femb, bias):
    bias16 = jnp.broadcast_to(bias.astype(jnp.float32), (16,))
    return _ffm_call(indices, femb, lin_tables, jnp.asarray(_META), bias16)


# speedup vs baseline: 27.4130x; 1.0273x over previous
import functools

import jax
import jax.numpy as jnp
import numpy as np
from jax import lax
from jax.experimental import pallas as pl
from jax.experimental.pallas import tpu as pltpu
from jax.experimental.pallas import tpu_sc as plsc

F = 26
B = 4096
V = 1000
D = 16

NC = 2
NS = 16
NW = NC * NS
BW = B // NW
P = 13
NPAIR = (F * (F - 1)) // 2
NBLK = NPAIR // P
META_COLS = 4 * NPAIR + 16


def _build_meta() -> np.ndarray:
    meta = np.zeros((META_COLS,), dtype=np.int32)
    p = 0
    for i in range(F - 1):
        for j in range(i + 1, F):
            meta[4 * p + 0] = i
            meta[4 * p + 1] = j
            p += 1
    return meta


_META = _build_meta()


def _ffm_body(idx_hbm, femb_hbm, lin_hbm, meta_hbm, bias_hbm, out_hbm,
              idxbuf, metabuf, biasbuf, bufA0, bufB0, bufA1, bufB1,
              linbuf, lacc, acc, res, sem0, sem1, sem2):
    wid = lax.axis_index("s") * NC + lax.axis_index("c")
    base = wid * BW
    rot = jnp.where(wid >= NBLK, wid - NBLK, wid)

    pltpu.sync_copy(idx_hbm.at[:, pl.ds(base, BW)], idxbuf)
    pltpu.sync_copy(meta_hbm, metabuf)
    pltpu.sync_copy(bias_hbm, biasbuf)

    zero16 = jnp.zeros((16,), jnp.float32)

    @pl.loop(0, BW)
    def _zero(b):
        acc[pl.ds(b * D, D)] = zero16

    def _fire(bb, bA, bB, sem):
        eb = bb + rot
        eb = jnp.where(eb >= NBLK, eb - NBLK, eb)
        for q in range(P):
            p = eb * P + q
            m = metabuf[pl.ds(4 * p, 16)]
            iA = m[0]
            jA = m[1]
            pltpu.async_copy(
                femb_hbm.at[iA, jA].at[idxbuf.at[iA]], bA.at[q], sem)
            pltpu.async_copy(
                femb_hbm.at[jA, iA].at[idxbuf.at[jA]], bB.at[q], sem)

    def _drain(bA, bB, sem):
        dummy = femb_hbm.at[0, pl.ds(0, P), pl.ds(0, BW), :]
        pltpu.make_async_copy(dummy, bA, sem).wait()
        pltpu.make_async_copy(dummy, bB, sem).wait()

    def _mac(bA, bB):
        @pl.loop(0, BW, unroll=4)
        def _m(b):
            a = acc[pl.ds(b * D, D)]
            for q in range(P):
                a = a + bA[q, b, :] * bB[q, b, :]
            acc[pl.ds(b * D, D)] = a

    for f in range(F):
        pltpu.async_copy(lin_hbm.at[f].at[idxbuf.at[f]], linbuf.at[f], sem2)

    _fire(0, bufA0, bufB0, sem0)

    @pl.loop(0, (NBLK - 1) // 2)
    def _piped(k):
        bb = k * 2
        _fire(bb + 1, bufA1, bufB1, sem1)
        _drain(bufA0, bufB0, sem0)
        _mac(bufA0, bufB0)
        _fire(bb + 2, bufA0, bufB0, sem0)
        _drain(bufA1, bufB1, sem1)
        _mac(bufA1, bufB1)

    _drain(bufA0, bufB0, sem0)
    _mac(bufA0, bufB0)

    pltpu.make_async_copy(lin_hbm.at[:, pl.ds(0, BW)], linbuf, sem2).wait()

    @pl.loop(0, BW // 16)
    def _linred(c):
        sl = pl.ds(c * 16, 16)
        v = linbuf[0, sl]
        for f in range(1, F):
            v = v + linbuf[f, sl]
        lacc[sl] = v

    lane = lax.iota(jnp.int32, 16)

    @pl.loop(0, BW // 16)
    def _final(c):
        x = biasbuf[:] + lacc[pl.ds(c * 16, 16)]
        for k in range(16):
            a = acc[pl.ds((c * 16 + k) * D, D)]
            s = jnp.zeros((16,), jnp.float32)
            for d in range(D):
                s = s + jnp.full((16,), a[d], jnp.float32)
            x = x + jnp.where(lane == k, s, 0.0)
        res[pl.ds(c * 16, 16)] = 1.0 / (1.0 + jnp.exp(-x))

    pltpu.sync_copy(res, out_hbm.at[pl.ds(base, BW)])


@jax.jit
def _ffm_call(indices, femb_flat, lin_pad, meta, bias16):
    mesh = plsc.VectorSubcoreMesh(core_axis_name="c", subcore_axis_name="s")
    return pl.kernel(
        _ffm_body,
        out_type=jax.ShapeDtypeStruct((B,), jnp.float32),
        mesh=mesh,
        compiler_params=pltpu.CompilerParams(use_tc_tiling_on_sc=False),
        scratch_types=[
            pltpu.VMEM((F, BW), jnp.int32),
            pltpu.VMEM((META_COLS,), jnp.int32),
            pltpu.VMEM((16,), jnp.float32),
            pltpu.VMEM((P, BW, D), jnp.float32),
            pltpu.VMEM((P, BW, D), jnp.float32),
            pltpu.VMEM((P, BW, D), jnp.float32),
            pltpu.VMEM((P, BW, D), jnp.float32),
            pltpu.VMEM((F, BW), jnp.float32),
            pltpu.VMEM((BW,), jnp.float32),
            pltpu.VMEM((BW * D,), jnp.float32),
            pltpu.VMEM((BW,), jnp.float32),
            pltpu.SemaphoreType.DMA,
            pltpu.SemaphoreType.DMA,
            pltpu.SemaphoreType.DMA,
        ],
    )(indices, femb_flat, lin_pad, meta, bias16)


def kernel(indices, lin_tables, femb, bias):
    bias16 = jnp.broadcast_to(bias.astype(jnp.float32), (16,))
    return _ffm_call(indices, femb, lin_tables, jnp.asarray(_META), bias16)
